# Initial kernel scaffold; baseline (speedup 1.0000x reference)
import jax
import jax.numpy as jnp
from jax.experimental import pallas as pl


def kernel(idx, wte, wpe):
    raise NotImplementedError("write your pallas kernel here")


# speedup vs baseline: 1.1568x; 1.1568x over previous
import functools

import jax
import jax.numpy as jnp
from jax import lax
from jax.experimental import pallas as pl
from jax.experimental.pallas import tpu as pltpu
from jax.experimental.pallas import tpu_sc as plsc

_NC, _NS, _L = 2, 16, 16
_NW = _NC * _NS


@functools.lru_cache(maxsize=None)
def _make_embed(B, T, V, D):
    TW = T // _NW
    n_vregs = D // _L

    mesh = plsc.VectorSubcoreMesh(core_axis_name="c", subcore_axis_name="s")

    @functools.partial(
        pl.kernel,
        out_type=jax.ShapeDtypeStruct((B * T, D), jnp.float32),
        mesh=mesh,
        scratch_types=[
            pltpu.VMEM((TW,), jnp.int32),
            pltpu.VMEM((TW, D), jnp.float32),
            pltpu.VMEM((TW, D), jnp.float32),
            pltpu.SemaphoreType.DMA,
        ],
    )
    def embed(idx_hbm, wte_hbm, wpe_hbm, out_hbm, idx_v, wpe_v, rows_v, sem):
        wid = lax.axis_index("s") * _NC + lax.axis_index("c")
        t0 = wid * TW
        pltpu.sync_copy(wpe_hbm.at[pl.ds(t0, TW)], wpe_v)
        for b in range(B):
            base = b * T + t0
            pltpu.sync_copy(idx_hbm.at[pl.ds(base, TW)], idx_v)
            pltpu.async_copy(wte_hbm.at[idx_v], rows_v, sem).wait()

            def row_body(r, carry):
                for c in range(n_vregs):
                    sl = pl.ds(c * _L, _L)
                    rows_v[r, sl] = rows_v[r, sl] + wpe_v[r, sl]
                return carry

            lax.fori_loop(0, TW, row_body, 0)
            pltpu.sync_copy(rows_v, out_hbm.at[pl.ds(base, TW)])

    return embed


def kernel(idx, wte, wpe):
    B, T = idx.shape
    V, D = wte.shape
    out = _make_embed(B, T, V, D)(idx.reshape(-1).astype(jnp.int32), wte, wpe)
    return out.reshape(B, T, D)
